# baseline (speedup 1.0000x reference)
import jax
import jax.numpy as jnp
from jax import lax
from jax.experimental import pallas as pl

NC = 16
D = 768
R = 1024
N_ROWS = 32768


def _pass1_body(x_ref, s_ref, np_ref, ctT_ref, cn_ref,
                w1_ref, w1b_ref, b1_ref, w2_ref, b2_ref, w3_ref, b3_ref,
                e1w_ref, e1b_ref, e2w_ref, e2b_ref,
                pred_ref, xc_ref, cnt_ref):
    t = pl.program_id(0)
    x = x_ref[...]
    npv = np_ref[...]
    sv = s_ref[...]

    h = x @ w1_ref[...] + npv * w1b_ref[...] + b1_ref[...]
    h = jnp.maximum(h, 0.0)
    h = jnp.maximum(h @ w2_ref[...] + b2_ref[...], 0.0)
    cp = h @ w3_ref[...] + b3_ref[...] + npv

    xx = jnp.sum(x * x, axis=1, keepdims=True)
    d = xx - 2.0 * (x @ ctT_ref[...]) + cn_ref[...]
    dmin = jnp.min(d, axis=1, keepdims=True)
    iota = lax.broadcasted_iota(jnp.int32, (R, NC), 1)
    xc = jnp.min(jnp.where(d == dmin, iota, NC), axis=1, keepdims=True)
    xc_ref[...] = xc

    onehot = (iota == xc).astype(jnp.float32)

    @pl.when(t == 0)
    def _init():
        cnt_ref[...] = jnp.zeros_like(cnt_ref)
    cnt_ref[...] += jnp.sum(onehot, axis=0, keepdims=True)

    w1sel = onehot @ e1w_ref[...]
    b1sel = onehot @ e1b_ref[...]
    w2sel = onehot @ e2w_ref[...]
    b2sel = onehot @ e2b_ref[...]

    hs = b1sel
    for k in range(6):
        hs = hs + cp[:, k:k + 1] * w1sel[:, k * 32:(k + 1) * 32]
    hs = jnp.maximum(hs, 0.0)

    c = []
    for j in range(6):
        cj = jnp.sum(hs * w2sel[:, j * 32:(j + 1) * 32], axis=1, keepdims=True)
        c.append(cj + b2sel[:, j:j + 1] + cp[:, j:j + 1])

    lin = -jnp.abs(c[0]) * sv + c[1] + npv
    logt = -jnp.abs(c[2]) * jnp.log10(jnp.abs(sv) + 1e-8) + c[3] + npv
    w0 = jnp.abs(c[4])
    w1 = jnp.abs(c[5])
    den = jnp.maximum(w0 + w1, 1e-12)
    pred_ref[...] = (w0 * lin + w1 * logt) / den


def _pass2_body(pred_ref, xc_ref, np_ref, cnt_ref, out_ref):
    xcv = xc_ref[...]
    valid = jnp.zeros(xcv.shape, dtype=jnp.bool_)
    for i in range(NC):
        ok_i = cnt_ref[0, i] >= 2.0
        valid = valid | ((xcv == i) & ok_i)
    out_ref[...] = jnp.where(valid, pred_ref[...], np_ref[...])


def kernel(x, s, naive_pred, centers, W1, b1, W2, b2, W3, b3,
           snW1, snb1, snW2, snb2):
    n = x.shape[0]
    s2 = s.reshape(n, 1)
    ctT = centers.T
    cn = jnp.sum(centers * centers, axis=1).reshape(1, NC)
    w1a = W1[:D]
    w1b = W1[D:D + 1]
    e1w = snW1.reshape(NC, 6 * 32)
    e2w = snW2.transpose(0, 2, 1).reshape(NC, 6 * 32)

    grid = (n // R,)
    pred, xc, cnt = pl.pallas_call(
        _pass1_body,
        grid=grid,
        in_specs=[
            pl.BlockSpec((R, D), lambda t: (t, 0)),
            pl.BlockSpec((R, 1), lambda t: (t, 0)),
            pl.BlockSpec((R, 1), lambda t: (t, 0)),
            pl.BlockSpec((D, NC), lambda t: (0, 0)),
            pl.BlockSpec((1, NC), lambda t: (0, 0)),
            pl.BlockSpec((D, 64), lambda t: (0, 0)),
            pl.BlockSpec((1, 64), lambda t: (0, 0)),
            pl.BlockSpec((1, 64), lambda t: (0, 0)),
            pl.BlockSpec((64, 32), lambda t: (0, 0)),
            pl.BlockSpec((1, 32), lambda t: (0, 0)),
            pl.BlockSpec((32, 6), lambda t: (0, 0)),
            pl.BlockSpec((1, 6), lambda t: (0, 0)),
            pl.BlockSpec((NC, 192), lambda t: (0, 0)),
            pl.BlockSpec((NC, 32), lambda t: (0, 0)),
            pl.BlockSpec((NC, 192), lambda t: (0, 0)),
            pl.BlockSpec((NC, 6), lambda t: (0, 0)),
        ],
        out_specs=[
            pl.BlockSpec((R, 1), lambda t: (t, 0)),
            pl.BlockSpec((R, 1), lambda t: (t, 0)),
            pl.BlockSpec((1, NC), lambda t: (0, 0)),
        ],
        out_shape=[
            jax.ShapeDtypeStruct((n, 1), jnp.float32),
            jax.ShapeDtypeStruct((n, 1), jnp.int32),
            jax.ShapeDtypeStruct((1, NC), jnp.float32),
        ],
    )(x, s2, naive_pred, ctT, cn, w1a, w1b, b1.reshape(1, 64),
      W2, b2.reshape(1, 32), W3, b3.reshape(1, 6),
      e1w, snb1, e2w, snb2)

    rows2 = n // 128
    out2 = pl.pallas_call(
        _pass2_body,
        in_specs=[
            pl.BlockSpec((rows2, 128), lambda: (0, 0)),
            pl.BlockSpec((rows2, 128), lambda: (0, 0)),
            pl.BlockSpec((rows2, 128), lambda: (0, 0)),
            pl.BlockSpec((1, NC), lambda: (0, 0)),
        ],
        out_specs=pl.BlockSpec((rows2, 128), lambda: (0, 0)),
        out_shape=jax.ShapeDtypeStruct((rows2, 128), jnp.float32),
    )(pred.reshape(rows2, 128), xc.reshape(rows2, 128),
      naive_pred.reshape(rows2, 128), cnt)
    return out2.reshape(n, 1)
